# baseline (speedup 1.0000x reference)
import functools
import math

import jax
import jax.numpy as jnp
from jax import lax
from jax.experimental import pallas as pl
from jax.experimental.pallas import tpu as pltpu
from jax.experimental.pallas import tpu_sc as plsc

_B = 32
_NA = 5
_H = 64
_W = 64
_HW = _H * _W
_G = 20
_C = 130
_CPA = 26
_NCLS = 20
_NBOX = 32
_ROWS_PER_STEP = 8192
_NSTEPS = _B * _HW // _ROWS_PER_STEP

_AW = (1.3221, 3.19275, 5.05587, 9.47112, 11.2364)
_AH = (1.73145, 4.00944, 8.09892, 4.84053, 10.0071)
_ATH = (0.0, 0.3927, 0.7854, 1.1781, 1.5708)

_COORD_SCALE = 5.0
_OBJECT_SCALE = 5.0
_CLASS_SCALE = 1.0
_THETA_SCALE = 5.0

_D_COORD = float(_B * _NA * 4 * _HW)
_D_CONF = float(_B * _NA * _HW)


def _a1c_body(x_ref, g_ref, t_ref,
              lt_ref, lco_ref, lcf_ref, lcl_ref, lth_ref, confacc):
    i = pl.program_id(0)

    @pl.when(i == 0)
    def _init():
        confacc[...] = jnp.zeros((1, 1), jnp.float32)

    x = x_ref[...]
    lane = lax.broadcasted_iota(jnp.int32, x.shape, 1)
    s = jax.nn.sigmoid(x)
    t = jnp.where(lane % _CPA == 4, 0.5 * s * s, 0.0)
    confacc[...] += jnp.reshape(jnp.sum(t), (1, 1))

    @pl.when(i == _NSTEPS - 1)
    def _final():
        densesum = jnp.sum(confacc[...])
        lt, lco, lcf, lcl, lth = _losses(g_ref[...], t_ref[...], densesum)
        lco_ref[...] = jnp.reshape(lco, (1, 1))
        lcf_ref[...] = jnp.reshape(lcf, (1, 1))
        lcl_ref[...] = jnp.reshape(lcl, (1, 1))
        lth_ref[...] = jnp.reshape(lth, (1, 1))
        lt_ref[...] = jnp.reshape(lt, (1, 1))


def _make_a1c(interpret=False):
    mk = lambda: jax.ShapeDtypeStruct((1, 1), jnp.float32)
    const2 = lambda i: (0, 0)
    const3 = lambda i: (0, 0, 0)
    return pl.pallas_call(
        _a1c_body,
        grid=(_NSTEPS,),
        in_specs=[
            pl.BlockSpec((_ROWS_PER_STEP, _C), lambda i: (i, 0)),
            pl.BlockSpec((_B, _G, _NBOX), const3),
            pl.BlockSpec((_B, _G, 6), const3),
        ],
        out_specs=[pl.BlockSpec((1, 1), const2)] * 5,
        out_shape=[mk() for _ in range(5)],
        scratch_shapes=[pltpu.VMEM((1, 1), jnp.float32)],
        interpret=interpret,
    )


def _sc_gather_kernel(table_hbm, t6_hbm, out_hbm,
                      tv, rows_v, choff_v, blk_v, box_v, sem):
    w = lax.axis_index("s") * 2 + lax.axis_index("c")
    pltpu.sync_copy(t6_hbm.at[w], tv)
    lanes = lax.iota(jnp.int32, 16)
    for h in range(2):
        sl = pl.ds(h * 16, 16)
        gx = tv[0, sl] * float(_W)
        gy = tv[1, sl] * float(_H)
        gth = tv[4, sl] * math.pi / 8
        q = gth * 0.25
        best = jnp.zeros((16,), jnp.int32)
        dbest = jnp.abs(q - jnp.float32(_ATH[0] * 0.25))
        for k in range(1, _NA):
            dk = jnp.abs(q - jnp.float32(_ATH[k] * 0.25))
            upd = dk < dbest
            best = jnp.where(upd, k, best)
            dbest = jnp.where(upd, dk, dbest)
        gi = jnp.clip(gx.astype(jnp.int32), 0, _W - 1)
        gj = jnp.clip(gy.astype(jnp.int32), 0, _H - 1)
        rows_v[sl] = gj * _W + gi
        choff_v[sl] = best * _CPA
    rms = []
    chs = []
    copies = []
    for k in range(_G):
        h, m = divmod(k, 16)
        sel = lanes == m
        row_s = jnp.sum(jnp.where(sel, rows_v[pl.ds(h * 16, 16)], 0))
        ch_s = jnp.sum(jnp.where(sel, choff_v[pl.ds(h * 16, 16)], 0))
        rm = jnp.bitwise_and(row_s, 7)
        row8 = pl.multiple_of(w * _HW + row_s - rm, 8)
        rms.append(rm)
        chs.append(ch_s)
        copies.append(pltpu.async_copy(
            table_hbm.at[pl.ds(row8, 8)], blk_v.at[k], sem))
    for cp in copies:
        cp.wait()
    mask2 = lanes < _CPA - 16
    for k in range(_G):
        rfull = jnp.full((16,), rms[k], jnp.int32)
        c1 = chs[k] + lanes
        c2 = chs[k] + 16 + lanes
        v1 = plsc.load_gather(blk_v.at[k], [rfull, c1])
        v2 = plsc.load_gather(blk_v.at[k], [rfull, c2], mask=mask2)
        box_v[k, pl.ds(0, 16)] = v1
        box_v[k, pl.ds(16, 16)] = v2
    pltpu.sync_copy(box_v, out_hbm.at[w])


def _make_sc_gather():
    mesh = plsc.VectorSubcoreMesh(core_axis_name="c", subcore_axis_name="s")
    return functools.partial(
        pl.kernel,
        mesh=mesh,
        out_type=jax.ShapeDtypeStruct((_B, _G, _NBOX), jnp.float32),
        scratch_types=[
            pltpu.VMEM((6, _NBOX), jnp.float32),
            pltpu.VMEM((_NBOX,), jnp.int32),
            pltpu.VMEM((_NBOX,), jnp.int32),
            pltpu.VMEM((_G, 8, _C), jnp.float32),
            pltpu.VMEM((_G, _NBOX), jnp.float32),
            pltpu.SemaphoreType.DMA,
        ],
        compiler_params=pltpu.CompilerParams(needs_layout_passes=False),
    )(_sc_gather_kernel)


def _smooth_l1(p, t):
    d = jnp.abs(p - t)
    return jnp.where(d < 1.0, 0.5 * d * d, d - 0.5)


def _losses(g, t, densesum):
    gx = t[:, :, 0] * _W
    gy = t[:, :, 1] * _H
    gw = t[:, :, 2] * _W
    gh = t[:, :, 3] * _H
    gth = t[:, :, 4] * math.pi / 8
    q = gth * 0.25
    best = jnp.zeros((_B, _G), jnp.int32)
    dbest = jnp.abs(q - jnp.float32(_ATH[0] * 0.25))
    cbest = jnp.full((_B, _G), _ATH[0] * 0.25, jnp.float32)
    aw = jnp.full((_B, _G), _AW[0], jnp.float32)
    ah = jnp.full((_B, _G), _AH[0], jnp.float32)
    ath = jnp.full((_B, _G), _ATH[0], jnp.float32)
    for k in range(1, _NA):
        dk = jnp.abs(q - jnp.float32(_ATH[k] * 0.25))
        upd = dk < dbest
        best = jnp.where(upd, k, best)
        dbest = jnp.where(upd, dk, dbest)
        cbest = jnp.where(upd, jnp.float32(_ATH[k] * 0.25), cbest)
        aw = jnp.where(upd, _AW[k], aw)
        ah = jnp.where(upd, _AH[k], ah)
        ath = jnp.where(upd, _ATH[k], ath)
    gi = jnp.clip(gx.astype(jnp.int32), 0, _W - 1)
    gj = jnp.clip(gy.astype(jnp.int32), 0, _H - 1)
    idx = gj * _W + gi
    l = best * _HW + idx
    eq = l[:, :, None] == l[:, None, :]
    jgt = (lax.broadcasted_iota(jnp.int32, (_B, _G, _G), 2)
           > lax.broadcasted_iota(jnp.int32, (_B, _G, _G), 1))
    dup = jnp.any(jnp.logical_and(eq, jgt), axis=2)
    live = jnp.where(dup, 0.0, 1.0)
    tx = gx - gi
    ty = gy - gj
    tw = jnp.log(jnp.maximum(gw, 1.0) / aw)
    th = jnp.log(jnp.maximum(gh, 1.0) / ah)
    tth = gth - ath
    tcf = jnp.cos(q - cbest)
    tcl = jnp.trunc(t[:, :, 5]).astype(jnp.int32)

    v = g[:, :, :_CPA]
    o0 = v[:, :, 0]
    o1 = v[:, :, 1]
    o2 = v[:, :, 2]
    o3 = v[:, :, 3]
    o4 = v[:, :, 4]
    o5 = v[:, :, 5]
    logits = v[:, :, 6:]

    coord_terms = (_smooth_l1(jax.nn.sigmoid(o0), tx)
                   + _smooth_l1(jax.nn.sigmoid(o1), ty)
                   + _smooth_l1(o2, tw)
                   + _smooth_l1(o3, th))
    coordsum = jnp.sum(live * coord_terms)

    conf = jax.nn.sigmoid(o4)
    confcorr = jnp.sum(live * (_smooth_l1(_OBJECT_SCALE * conf,
                                          _OBJECT_SCALE * tcf)
                               - 0.5 * conf * conf))
    nmask = jnp.sum(live)
    thetasum = jnp.sum(live * _smooth_l1(o5, tth))

    m = jnp.max(logits, axis=2)
    lse = m + jnp.log(jnp.sum(jnp.exp(logits - m[:, :, None]), axis=2))
    cls_iota = lax.broadcasted_iota(jnp.int32, (_B, _G, _NCLS), 2)
    ll = jnp.sum(jnp.where(cls_iota == tcl[:, :, None], logits, 0.0), axis=2)
    clssum = jnp.sum(live * (lse - ll))

    loss_coord = _COORD_SCALE * coordsum / _D_COORD
    loss_conf = (densesum + confcorr) / _D_CONF
    loss_cls = _CLASS_SCALE * 2.0 * clssum / nmask
    loss_theta = _THETA_SCALE * thetasum / nmask
    loss_tot = loss_coord + loss_conf + loss_cls + loss_theta
    return loss_tot, loss_coord, loss_conf, loss_cls, loss_theta


_a1c = _make_a1c()


def kernel(output, target):
    table = jnp.transpose(output, (0, 2, 3, 1)).reshape(_B * _HW, _C)
    t6 = jnp.pad(jnp.transpose(target, (0, 2, 1)), ((0, 0), (0, 0), (0, 12)))
    g = _make_sc_gather()(table, t6)
    lt, lco, lcf, lcl, lth = _a1c(table, g, target)
    return (lt.reshape(()), lco.reshape(()), lcf.reshape(()),
            lcl.reshape(()), lth.reshape(()))


# speedup vs baseline: 1.4364x; 1.4364x over previous
import functools
import math

import jax
import jax.numpy as jnp
from jax import lax
from jax.experimental import pallas as pl
from jax.experimental.pallas import tpu as pltpu
from jax.experimental.pallas import tpu_sc as plsc

_B = 32
_NA = 5
_H = 64
_W = 64
_HW = _H * _W
_G = 20
_C = 130
_CPA = 26
_NCLS = 20
_NBOX = 32
_ROWS_PER_STEP = 8192
_NSTEPS = _B * _HW // _ROWS_PER_STEP

_AW = (1.3221, 3.19275, 5.05587, 9.47112, 11.2364)
_AH = (1.73145, 4.00944, 8.09892, 4.84053, 10.0071)
_ATH = (0.0, 0.3927, 0.7854, 1.1781, 1.5708)

_COORD_SCALE = 5.0
_OBJECT_SCALE = 5.0
_CLASS_SCALE = 1.0
_THETA_SCALE = 5.0

_D_COORD = float(_B * _NA * 4 * _HW)
_D_CONF = float(_B * _NA * _HW)


def _a1_body(x_ref, confsum_ref):
    i = pl.program_id(0)

    @pl.when(i == 0)
    def _init():
        confsum_ref[...] = jnp.zeros((1, 1), jnp.float32)

    x = x_ref[...]
    lane = lax.broadcasted_iota(jnp.int32, x.shape, 1)
    s = jax.nn.sigmoid(x)
    t = jnp.where(lane % _CPA == 4, 0.5 * s * s, 0.0)
    confsum_ref[...] += jnp.reshape(jnp.sum(t), (1, 1))


def _make_a1(interpret=False):
    return pl.pallas_call(
        _a1_body,
        grid=(_NSTEPS,),
        in_specs=[pl.BlockSpec((_ROWS_PER_STEP, 128), lambda i: (i, 0))],
        out_specs=pl.BlockSpec((1, 1), lambda i: (0, 0)),
        out_shape=jax.ShapeDtypeStruct((1, 1), jnp.float32),
        interpret=interpret,
    )


def _c_body(g_ref, t_ref, confsum_ref,
            lt_ref, lco_ref, lcf_ref, lcl_ref, lth_ref):
    densesum = jnp.sum(confsum_ref[...])
    lt, lco, lcf, lcl, lth = _losses(g_ref[...], t_ref[...], densesum)
    lco_ref[...] = jnp.reshape(lco, (1, 1))
    lcf_ref[...] = jnp.reshape(lcf, (1, 1))
    lcl_ref[...] = jnp.reshape(lcl, (1, 1))
    lth_ref[...] = jnp.reshape(lth, (1, 1))
    lt_ref[...] = jnp.reshape(lt, (1, 1))


def _make_stage_c(interpret=False):
    mk = lambda: jax.ShapeDtypeStruct((1, 1), jnp.float32)
    return pl.pallas_call(
        _c_body,
        out_shape=[mk() for _ in range(5)],
        interpret=interpret,
    )


def _sc_gather_kernel(table_hbm, t6_hbm, out_hbm,
                      tv, rows_v, choff_v, blk_v, box_v, sem):
    w = lax.axis_index("s") * 2 + lax.axis_index("c")
    pltpu.sync_copy(t6_hbm.at[w], tv)
    lanes = lax.iota(jnp.int32, 16)
    for h in range(2):
        sl = pl.ds(h * 16, 16)
        gx = tv[0, sl] * float(_W)
        gy = tv[1, sl] * float(_H)
        gth = tv[4, sl] * math.pi / 8
        q = gth * 0.25
        best = jnp.zeros((16,), jnp.int32)
        dbest = jnp.abs(q - jnp.float32(_ATH[0] * 0.25))
        for k in range(1, _NA):
            dk = jnp.abs(q - jnp.float32(_ATH[k] * 0.25))
            upd = dk < dbest
            best = jnp.where(upd, k, best)
            dbest = jnp.where(upd, dk, dbest)
        gi = jnp.clip(gx.astype(jnp.int32), 0, _W - 1)
        gj = jnp.clip(gy.astype(jnp.int32), 0, _H - 1)
        rows_v[sl] = gj * _W + gi
        choff_v[sl] = best * _CPA
    rms = []
    chs = []
    copies = []
    for k in range(_G):
        h, m = divmod(k, 16)
        sel = lanes == m
        row_s = jnp.sum(jnp.where(sel, rows_v[pl.ds(h * 16, 16)], 0))
        ch_s = jnp.sum(jnp.where(sel, choff_v[pl.ds(h * 16, 16)], 0))
        rm = jnp.bitwise_and(row_s, 7)
        row8 = pl.multiple_of(w * _HW + row_s - rm, 8)
        rms.append(rm)
        chs.append(ch_s)
        copies.append(pltpu.async_copy(
            table_hbm.at[pl.ds(row8, 8)], blk_v.at[k], sem))
    for cp in copies:
        cp.wait()
    mask2 = lanes < _CPA - 16
    for k in range(_G):
        rfull = jnp.full((16,), rms[k], jnp.int32)
        c1 = chs[k] + lanes
        c2 = chs[k] + 16 + lanes
        v1 = plsc.load_gather(blk_v.at[k], [rfull, c1])
        v2 = plsc.load_gather(blk_v.at[k], [rfull, c2], mask=mask2)
        box_v[k, pl.ds(0, 16)] = v1
        box_v[k, pl.ds(16, 16)] = v2
    pltpu.sync_copy(box_v, out_hbm.at[w])


def _make_sc_gather():
    mesh = plsc.VectorSubcoreMesh(core_axis_name="c", subcore_axis_name="s")
    return functools.partial(
        pl.kernel,
        mesh=mesh,
        out_type=jax.ShapeDtypeStruct((_B, _G, _NBOX), jnp.float32),
        scratch_types=[
            pltpu.VMEM((6, _NBOX), jnp.float32),
            pltpu.VMEM((_NBOX,), jnp.int32),
            pltpu.VMEM((_NBOX,), jnp.int32),
            pltpu.VMEM((_G, 8, _C), jnp.float32),
            pltpu.VMEM((_G, _NBOX), jnp.float32),
            pltpu.SemaphoreType.DMA,
        ],
        compiler_params=pltpu.CompilerParams(needs_layout_passes=False),
    )(_sc_gather_kernel)


def _smooth_l1(p, t):
    d = jnp.abs(p - t)
    return jnp.where(d < 1.0, 0.5 * d * d, d - 0.5)


def _losses(g, t, densesum):
    gx = t[:, :, 0] * _W
    gy = t[:, :, 1] * _H
    gw = t[:, :, 2] * _W
    gh = t[:, :, 3] * _H
    gth = t[:, :, 4] * math.pi / 8
    q = gth * 0.25
    best = jnp.zeros((_B, _G), jnp.int32)
    dbest = jnp.abs(q - jnp.float32(_ATH[0] * 0.25))
    cbest = jnp.full((_B, _G), _ATH[0] * 0.25, jnp.float32)
    aw = jnp.full((_B, _G), _AW[0], jnp.float32)
    ah = jnp.full((_B, _G), _AH[0], jnp.float32)
    ath = jnp.full((_B, _G), _ATH[0], jnp.float32)
    for k in range(1, _NA):
        dk = jnp.abs(q - jnp.float32(_ATH[k] * 0.25))
        upd = dk < dbest
        best = jnp.where(upd, k, best)
        dbest = jnp.where(upd, dk, dbest)
        cbest = jnp.where(upd, jnp.float32(_ATH[k] * 0.25), cbest)
        aw = jnp.where(upd, _AW[k], aw)
        ah = jnp.where(upd, _AH[k], ah)
        ath = jnp.where(upd, _ATH[k], ath)
    gi = jnp.clip(gx.astype(jnp.int32), 0, _W - 1)
    gj = jnp.clip(gy.astype(jnp.int32), 0, _H - 1)
    idx = gj * _W + gi
    l = best * _HW + idx
    eq = l[:, :, None] == l[:, None, :]
    jgt = (lax.broadcasted_iota(jnp.int32, (_B, _G, _G), 2)
           > lax.broadcasted_iota(jnp.int32, (_B, _G, _G), 1))
    dup = jnp.any(jnp.logical_and(eq, jgt), axis=2)
    live = jnp.where(dup, 0.0, 1.0)
    tx = gx - gi
    ty = gy - gj
    tw = jnp.log(jnp.maximum(gw, 1.0) / aw)
    th = jnp.log(jnp.maximum(gh, 1.0) / ah)
    tth = gth - ath
    tcf = jnp.cos(q - cbest)
    tcl = jnp.trunc(t[:, :, 5]).astype(jnp.int32)

    v = g[:, :, :_CPA]
    o0 = v[:, :, 0]
    o1 = v[:, :, 1]
    o2 = v[:, :, 2]
    o3 = v[:, :, 3]
    o4 = v[:, :, 4]
    o5 = v[:, :, 5]
    logits = v[:, :, 6:]

    coord_terms = (_smooth_l1(jax.nn.sigmoid(o0), tx)
                   + _smooth_l1(jax.nn.sigmoid(o1), ty)
                   + _smooth_l1(o2, tw)
                   + _smooth_l1(o3, th))
    coordsum = jnp.sum(live * coord_terms)

    conf = jax.nn.sigmoid(o4)
    confcorr = jnp.sum(live * (_smooth_l1(_OBJECT_SCALE * conf,
                                          _OBJECT_SCALE * tcf)
                               - 0.5 * conf * conf))
    nmask = jnp.sum(live)
    thetasum = jnp.sum(live * _smooth_l1(o5, tth))

    m = jnp.max(logits, axis=2)
    lse = m + jnp.log(jnp.sum(jnp.exp(logits - m[:, :, None]), axis=2))
    cls_iota = lax.broadcasted_iota(jnp.int32, (_B, _G, _NCLS), 2)
    ll = jnp.sum(jnp.where(cls_iota == tcl[:, :, None], logits, 0.0), axis=2)
    clssum = jnp.sum(live * (lse - ll))

    loss_coord = _COORD_SCALE * coordsum / _D_COORD
    loss_conf = (densesum + confcorr) / _D_CONF
    loss_cls = _CLASS_SCALE * 2.0 * clssum / nmask
    loss_theta = _THETA_SCALE * thetasum / nmask
    loss_tot = loss_coord + loss_conf + loss_cls + loss_theta
    return loss_tot, loss_coord, loss_conf, loss_cls, loss_theta


_a1 = _make_a1()
_stage_c = _make_stage_c()


def kernel(output, target):
    table = jnp.transpose(output, (0, 2, 3, 1)).reshape(_B * _HW, _C)
    t6 = jnp.pad(jnp.transpose(target, (0, 2, 1)), ((0, 0), (0, 0), (0, 12)))
    g = _make_sc_gather()(table, t6)
    confsum = _a1(table)
    lt, lco, lcf, lcl, lth = _stage_c(g, target, confsum)
    return (lt.reshape(()), lco.reshape(()), lcf.reshape(()),
            lcl.reshape(()), lth.reshape(()))


# speedup vs baseline: 1.4524x; 1.0111x over previous
import functools
import math

import jax
import jax.numpy as jnp
from jax import lax
from jax.experimental import pallas as pl
from jax.experimental.pallas import tpu as pltpu
from jax.experimental.pallas import tpu_sc as plsc

_B = 32
_NA = 5
_H = 64
_W = 64
_HW = _H * _W
_G = 20
_C = 130
_CPA = 26
_NCLS = 20
_NBOX = 32
_ROWS_PER_STEP = 8192
_NSTEPS = _B * _HW // _ROWS_PER_STEP

_AW = (1.3221, 3.19275, 5.05587, 9.47112, 11.2364)
_AH = (1.73145, 4.00944, 8.09892, 4.84053, 10.0071)
_ATH = (0.0, 0.3927, 0.7854, 1.1781, 1.5708)

_COORD_SCALE = 5.0
_OBJECT_SCALE = 5.0
_CLASS_SCALE = 1.0
_THETA_SCALE = 5.0

_D_COORD = float(_B * _NA * 4 * _HW)
_D_CONF = float(_B * _NA * _HW)


def _a1_body(x_ref, confsum_ref):
    i = pl.program_id(0)

    @pl.when(i == 0)
    def _init():
        confsum_ref[...] = jnp.zeros((1, 1), jnp.float32)

    x = x_ref[...]
    lane = lax.broadcasted_iota(jnp.int32, x.shape, 1)
    t1 = 1.0 + jnp.exp(-x)
    r = 1.0 / t1
    t = jnp.where(lane % _CPA == 4, r * r, 0.0)
    confsum_ref[...] += jnp.reshape(0.5 * jnp.sum(t), (1, 1))


def _make_a1(interpret=False):
    return pl.pallas_call(
        _a1_body,
        grid=(_NSTEPS,),
        in_specs=[pl.BlockSpec((_ROWS_PER_STEP, 128), lambda i: (i, 0))],
        out_specs=pl.BlockSpec((1, 1), lambda i: (0, 0)),
        out_shape=jax.ShapeDtypeStruct((1, 1), jnp.float32),
        interpret=interpret,
    )


def _c_body(g_ref, t_ref, confsum_ref,
            lt_ref, lco_ref, lcf_ref, lcl_ref, lth_ref):
    densesum = jnp.sum(confsum_ref[...])
    lt, lco, lcf, lcl, lth = _losses(g_ref[...], t_ref[...], densesum)
    lco_ref[...] = jnp.reshape(lco, (1, 1))
    lcf_ref[...] = jnp.reshape(lcf, (1, 1))
    lcl_ref[...] = jnp.reshape(lcl, (1, 1))
    lth_ref[...] = jnp.reshape(lth, (1, 1))
    lt_ref[...] = jnp.reshape(lt, (1, 1))


def _make_stage_c(interpret=False):
    mk = lambda: jax.ShapeDtypeStruct((1, 1), jnp.float32)
    return pl.pallas_call(
        _c_body,
        out_shape=[mk() for _ in range(5)],
        interpret=interpret,
    )


def _sc_gather_kernel(table_hbm, t6_hbm, out_hbm,
                      tv, rows_v, choff_v, blk_v, box_v, sem):
    w = lax.axis_index("s") * 2 + lax.axis_index("c")
    pltpu.sync_copy(t6_hbm.at[w], tv)
    lanes = lax.iota(jnp.int32, 16)
    for h in range(2):
        sl = pl.ds(h * 16, 16)
        gx = tv[0, sl] * float(_W)
        gy = tv[1, sl] * float(_H)
        gth = tv[4, sl] * math.pi / 8
        q = gth * 0.25
        best = jnp.zeros((16,), jnp.int32)
        dbest = jnp.abs(q - jnp.float32(_ATH[0] * 0.25))
        for k in range(1, _NA):
            dk = jnp.abs(q - jnp.float32(_ATH[k] * 0.25))
            upd = dk < dbest
            best = jnp.where(upd, k, best)
            dbest = jnp.where(upd, dk, dbest)
        gi = jnp.clip(gx.astype(jnp.int32), 0, _W - 1)
        gj = jnp.clip(gy.astype(jnp.int32), 0, _H - 1)
        rows_v[sl] = gj * _W + gi
        choff_v[sl] = best * _CPA
    rms = []
    chs = []
    copies = []
    for k in range(_G):
        h, m = divmod(k, 16)
        sel = lanes == m
        row_s = jnp.sum(jnp.where(sel, rows_v[pl.ds(h * 16, 16)], 0))
        ch_s = jnp.sum(jnp.where(sel, choff_v[pl.ds(h * 16, 16)], 0))
        rm = jnp.bitwise_and(row_s, 7)
        row8 = pl.multiple_of(w * _HW + row_s - rm, 8)
        rms.append(rm)
        chs.append(ch_s)
        copies.append(pltpu.async_copy(
            table_hbm.at[pl.ds(row8, 8)], blk_v.at[k], sem))
    for cp in copies:
        cp.wait()
    mask2 = lanes < _CPA - 16
    for k in range(_G):
        rfull = jnp.full((16,), rms[k], jnp.int32)
        c1 = chs[k] + lanes
        c2 = chs[k] + 16 + lanes
        v1 = plsc.load_gather(blk_v.at[k], [rfull, c1])
        v2 = plsc.load_gather(blk_v.at[k], [rfull, c2], mask=mask2)
        box_v[k, pl.ds(0, 16)] = v1
        box_v[k, pl.ds(16, 16)] = v2
    pltpu.sync_copy(box_v, out_hbm.at[w])


def _make_sc_gather():
    mesh = plsc.VectorSubcoreMesh(core_axis_name="c", subcore_axis_name="s")
    return functools.partial(
        pl.kernel,
        mesh=mesh,
        out_type=jax.ShapeDtypeStruct((_B, _G, _NBOX), jnp.float32),
        scratch_types=[
            pltpu.VMEM((6, _NBOX), jnp.float32),
            pltpu.VMEM((_NBOX,), jnp.int32),
            pltpu.VMEM((_NBOX,), jnp.int32),
            pltpu.VMEM((_G, 8, _C), jnp.float32),
            pltpu.VMEM((_G, _NBOX), jnp.float32),
            pltpu.SemaphoreType.DMA,
        ],
        compiler_params=pltpu.CompilerParams(needs_layout_passes=False),
    )(_sc_gather_kernel)


def _smooth_l1(p, t):
    d = jnp.abs(p - t)
    return jnp.where(d < 1.0, 0.5 * d * d, d - 0.5)


def _losses(g, t, densesum):
    gx = t[:, :, 0] * _W
    gy = t[:, :, 1] * _H
    gw = t[:, :, 2] * _W
    gh = t[:, :, 3] * _H
    gth = t[:, :, 4] * math.pi / 8
    q = gth * 0.25
    best = jnp.zeros((_B, _G), jnp.int32)
    dbest = jnp.abs(q - jnp.float32(_ATH[0] * 0.25))
    cbest = jnp.full((_B, _G), _ATH[0] * 0.25, jnp.float32)
    aw = jnp.full((_B, _G), _AW[0], jnp.float32)
    ah = jnp.full((_B, _G), _AH[0], jnp.float32)
    ath = jnp.full((_B, _G), _ATH[0], jnp.float32)
    for k in range(1, _NA):
        dk = jnp.abs(q - jnp.float32(_ATH[k] * 0.25))
        upd = dk < dbest
        best = jnp.where(upd, k, best)
        dbest = jnp.where(upd, dk, dbest)
        cbest = jnp.where(upd, jnp.float32(_ATH[k] * 0.25), cbest)
        aw = jnp.where(upd, _AW[k], aw)
        ah = jnp.where(upd, _AH[k], ah)
        ath = jnp.where(upd, _ATH[k], ath)
    gi = jnp.clip(gx.astype(jnp.int32), 0, _W - 1)
    gj = jnp.clip(gy.astype(jnp.int32), 0, _H - 1)
    idx = gj * _W + gi
    l = best * _HW + idx
    eq = l[:, :, None] == l[:, None, :]
    jgt = (lax.broadcasted_iota(jnp.int32, (_B, _G, _G), 2)
           > lax.broadcasted_iota(jnp.int32, (_B, _G, _G), 1))
    dup = jnp.any(jnp.logical_and(eq, jgt), axis=2)
    live = jnp.where(dup, 0.0, 1.0)
    tx = gx - gi
    ty = gy - gj
    tw = jnp.log(jnp.maximum(gw, 1.0) / aw)
    th = jnp.log(jnp.maximum(gh, 1.0) / ah)
    tth = gth - ath
    tcf = jnp.cos(q - cbest)
    tcl = jnp.trunc(t[:, :, 5]).astype(jnp.int32)

    v = g[:, :, :_CPA]
    o0 = v[:, :, 0]
    o1 = v[:, :, 1]
    o2 = v[:, :, 2]
    o3 = v[:, :, 3]
    o4 = v[:, :, 4]
    o5 = v[:, :, 5]
    logits = v[:, :, 6:]

    coord_terms = (_smooth_l1(jax.nn.sigmoid(o0), tx)
                   + _smooth_l1(jax.nn.sigmoid(o1), ty)
                   + _smooth_l1(o2, tw)
                   + _smooth_l1(o3, th))
    coordsum = jnp.sum(live * coord_terms)

    conf = jax.nn.sigmoid(o4)
    confcorr = jnp.sum(live * (_smooth_l1(_OBJECT_SCALE * conf,
                                          _OBJECT_SCALE * tcf)
                               - 0.5 * conf * conf))
    nmask = jnp.sum(live)
    thetasum = jnp.sum(live * _smooth_l1(o5, tth))

    m = jnp.max(logits, axis=2)
    lse = m + jnp.log(jnp.sum(jnp.exp(logits - m[:, :, None]), axis=2))
    cls_iota = lax.broadcasted_iota(jnp.int32, (_B, _G, _NCLS), 2)
    ll = jnp.sum(jnp.where(cls_iota == tcl[:, :, None], logits, 0.0), axis=2)
    clssum = jnp.sum(live * (lse - ll))

    loss_coord = _COORD_SCALE * coordsum / _D_COORD
    loss_conf = (densesum + confcorr) / _D_CONF
    loss_cls = _CLASS_SCALE * 2.0 * clssum / nmask
    loss_theta = _THETA_SCALE * thetasum / nmask
    loss_tot = loss_coord + loss_conf + loss_cls + loss_theta
    return loss_tot, loss_coord, loss_conf, loss_cls, loss_theta


_a1 = _make_a1()
_stage_c = _make_stage_c()


def kernel(output, target):
    table = jnp.transpose(output, (0, 2, 3, 1)).reshape(_B * _HW, _C)
    t6 = jnp.pad(jnp.transpose(target, (0, 2, 1)), ((0, 0), (0, 0), (0, 12)))
    g = _make_sc_gather()(table, t6)
    confsum = _a1(table)
    lt, lco, lcf, lcl, lth = _stage_c(g, target, confsum)
    return (lt.reshape(()), lco.reshape(()), lcf.reshape(()),
            lcl.reshape(()), lth.reshape(()))


# speedup vs baseline: 1.5728x; 1.0829x over previous
import functools
import math

import jax
import jax.numpy as jnp
from jax import lax
from jax.experimental import pallas as pl
from jax.experimental.pallas import tpu as pltpu
from jax.experimental.pallas import tpu_sc as plsc

_B = 32
_NA = 5
_H = 64
_W = 64
_HW = _H * _W
_G = 20
_C = 130
_CPA = 26
_NCLS = 20
_NBOX = 32
_ROWS_PER_STEP = 16384
_NSTEPS = _B * _HW // _ROWS_PER_STEP

_AW = (1.3221, 3.19275, 5.05587, 9.47112, 11.2364)
_AH = (1.73145, 4.00944, 8.09892, 4.84053, 10.0071)
_ATH = (0.0, 0.3927, 0.7854, 1.1781, 1.5708)

_COORD_SCALE = 5.0
_OBJECT_SCALE = 5.0
_CLASS_SCALE = 1.0
_THETA_SCALE = 5.0

_D_COORD = float(_B * _NA * 4 * _HW)
_D_CONF = float(_B * _NA * _HW)


def _a1_body(x_ref, confsum_ref):
    i = pl.program_id(0)

    @pl.when(i == 0)
    def _init():
        confsum_ref[...] = jnp.zeros((1, 1), jnp.float32)

    x = x_ref[...]
    lane = lax.broadcasted_iota(jnp.int32, x.shape, 1)
    t1 = 1.0 + jnp.exp(-x)
    r = 1.0 / t1
    t = jnp.where(lane % _CPA == 4, r * r, 0.0)
    confsum_ref[...] += jnp.reshape(0.5 * jnp.sum(t), (1, 1))


def _make_a1(interpret=False):
    return pl.pallas_call(
        _a1_body,
        grid=(_NSTEPS,),
        in_specs=[pl.BlockSpec((_ROWS_PER_STEP, 128), lambda i: (i, 0))],
        out_specs=pl.BlockSpec((1, 1), lambda i: (0, 0)),
        out_shape=jax.ShapeDtypeStruct((1, 1), jnp.float32),
        interpret=interpret,
    )


def _c_body(g_ref, t_ref, confsum_ref,
            lt_ref, lco_ref, lcf_ref, lcl_ref, lth_ref):
    densesum = jnp.sum(confsum_ref[...])
    lt, lco, lcf, lcl, lth = _losses(g_ref[...], t_ref[...], densesum)
    lco_ref[...] = jnp.reshape(lco, (1, 1))
    lcf_ref[...] = jnp.reshape(lcf, (1, 1))
    lcl_ref[...] = jnp.reshape(lcl, (1, 1))
    lth_ref[...] = jnp.reshape(lth, (1, 1))
    lt_ref[...] = jnp.reshape(lt, (1, 1))


def _make_stage_c(interpret=False):
    mk = lambda: jax.ShapeDtypeStruct((1, 1), jnp.float32)
    return pl.pallas_call(
        _c_body,
        out_shape=[mk() for _ in range(5)],
        interpret=interpret,
    )


def _sc_gather_kernel(table_hbm, t6_hbm, out_hbm,
                      tv, rows_v, choff_v, blk_v, box_v, sem):
    w = lax.axis_index("s") * 2 + lax.axis_index("c")
    pltpu.sync_copy(t6_hbm.at[w], tv)
    lanes = lax.iota(jnp.int32, 16)
    for h in range(2):
        sl = pl.ds(h * 16, 16)
        gx = tv[0, sl] * float(_W)
        gy = tv[1, sl] * float(_H)
        gth = tv[4, sl] * math.pi / 8
        q = gth * 0.25
        best = jnp.zeros((16,), jnp.int32)
        dbest = jnp.abs(q - jnp.float32(_ATH[0] * 0.25))
        for k in range(1, _NA):
            dk = jnp.abs(q - jnp.float32(_ATH[k] * 0.25))
            upd = dk < dbest
            best = jnp.where(upd, k, best)
            dbest = jnp.where(upd, dk, dbest)
        gi = jnp.clip(gx.astype(jnp.int32), 0, _W - 1)
        gj = jnp.clip(gy.astype(jnp.int32), 0, _H - 1)
        rows_v[sl] = gj * _W + gi
        choff_v[sl] = best * _CPA
    rms = []
    chs = []
    copies = []
    for k in range(_G):
        h, m = divmod(k, 16)
        sel = lanes == m
        row_s = jnp.sum(jnp.where(sel, rows_v[pl.ds(h * 16, 16)], 0))
        ch_s = jnp.sum(jnp.where(sel, choff_v[pl.ds(h * 16, 16)], 0))
        rm = jnp.bitwise_and(row_s, 7)
        row8 = pl.multiple_of(w * _HW + row_s - rm, 8)
        rms.append(rm)
        chs.append(ch_s)
        copies.append(pltpu.async_copy(
            table_hbm.at[pl.ds(row8, 8)], blk_v.at[k], sem))
    for cp in copies:
        cp.wait()
    mask2 = lanes < _CPA - 16
    for k in range(_G):
        rfull = jnp.full((16,), rms[k], jnp.int32)
        c1 = chs[k] + lanes
        c2 = chs[k] + 16 + lanes
        v1 = plsc.load_gather(blk_v.at[k], [rfull, c1])
        v2 = plsc.load_gather(blk_v.at[k], [rfull, c2], mask=mask2)
        box_v[k, pl.ds(0, 16)] = v1
        box_v[k, pl.ds(16, 16)] = v2
    pltpu.sync_copy(box_v, out_hbm.at[w])


def _make_sc_gather():
    mesh = plsc.VectorSubcoreMesh(core_axis_name="c", subcore_axis_name="s")
    return functools.partial(
        pl.kernel,
        mesh=mesh,
        out_type=jax.ShapeDtypeStruct((_B, _G, _NBOX), jnp.float32),
        scratch_types=[
            pltpu.VMEM((6, _NBOX), jnp.float32),
            pltpu.VMEM((_NBOX,), jnp.int32),
            pltpu.VMEM((_NBOX,), jnp.int32),
            pltpu.VMEM((_G, 8, _C), jnp.float32),
            pltpu.VMEM((_G, _NBOX), jnp.float32),
            pltpu.SemaphoreType.DMA,
        ],
        compiler_params=pltpu.CompilerParams(needs_layout_passes=False),
    )(_sc_gather_kernel)


def _smooth_l1(p, t):
    d = jnp.abs(p - t)
    return jnp.where(d < 1.0, 0.5 * d * d, d - 0.5)


def _losses(g, t, densesum):
    gx = t[:, :, 0] * _W
    gy = t[:, :, 1] * _H
    gw = t[:, :, 2] * _W
    gh = t[:, :, 3] * _H
    gth = t[:, :, 4] * math.pi / 8
    q = gth * 0.25
    best = jnp.zeros((_B, _G), jnp.int32)
    dbest = jnp.abs(q - jnp.float32(_ATH[0] * 0.25))
    cbest = jnp.full((_B, _G), _ATH[0] * 0.25, jnp.float32)
    aw = jnp.full((_B, _G), _AW[0], jnp.float32)
    ah = jnp.full((_B, _G), _AH[0], jnp.float32)
    ath = jnp.full((_B, _G), _ATH[0], jnp.float32)
    for k in range(1, _NA):
        dk = jnp.abs(q - jnp.float32(_ATH[k] * 0.25))
        upd = dk < dbest
        best = jnp.where(upd, k, best)
        dbest = jnp.where(upd, dk, dbest)
        cbest = jnp.where(upd, jnp.float32(_ATH[k] * 0.25), cbest)
        aw = jnp.where(upd, _AW[k], aw)
        ah = jnp.where(upd, _AH[k], ah)
        ath = jnp.where(upd, _ATH[k], ath)
    gi = jnp.clip(gx.astype(jnp.int32), 0, _W - 1)
    gj = jnp.clip(gy.astype(jnp.int32), 0, _H - 1)
    idx = gj * _W + gi
    l = best * _HW + idx
    eq = l[:, :, None] == l[:, None, :]
    jgt = (lax.broadcasted_iota(jnp.int32, (_B, _G, _G), 2)
           > lax.broadcasted_iota(jnp.int32, (_B, _G, _G), 1))
    dup = jnp.any(jnp.logical_and(eq, jgt), axis=2)
    live = jnp.where(dup, 0.0, 1.0)
    tx = gx - gi
    ty = gy - gj
    tw = jnp.log(jnp.maximum(gw, 1.0) / aw)
    th = jnp.log(jnp.maximum(gh, 1.0) / ah)
    tth = gth - ath
    tcf = jnp.cos(q - cbest)
    tcl = jnp.trunc(t[:, :, 5]).astype(jnp.int32)

    v = g[:, :, :_CPA]
    o0 = v[:, :, 0]
    o1 = v[:, :, 1]
    o2 = v[:, :, 2]
    o3 = v[:, :, 3]
    o4 = v[:, :, 4]
    o5 = v[:, :, 5]
    logits = v[:, :, 6:]

    coord_terms = (_smooth_l1(jax.nn.sigmoid(o0), tx)
                   + _smooth_l1(jax.nn.sigmoid(o1), ty)
                   + _smooth_l1(o2, tw)
                   + _smooth_l1(o3, th))
    coordsum = jnp.sum(live * coord_terms)

    conf = jax.nn.sigmoid(o4)
    confcorr = jnp.sum(live * (_smooth_l1(_OBJECT_SCALE * conf,
                                          _OBJECT_SCALE * tcf)
                               - 0.5 * conf * conf))
    nmask = jnp.sum(live)
    thetasum = jnp.sum(live * _smooth_l1(o5, tth))

    m = jnp.max(logits, axis=2)
    lse = m + jnp.log(jnp.sum(jnp.exp(logits - m[:, :, None]), axis=2))
    cls_iota = lax.broadcasted_iota(jnp.int32, (_B, _G, _NCLS), 2)
    ll = jnp.sum(jnp.where(cls_iota == tcl[:, :, None], logits, 0.0), axis=2)
    clssum = jnp.sum(live * (lse - ll))

    loss_coord = _COORD_SCALE * coordsum / _D_COORD
    loss_conf = (densesum + confcorr) / _D_CONF
    loss_cls = _CLASS_SCALE * 2.0 * clssum / nmask
    loss_theta = _THETA_SCALE * thetasum / nmask
    loss_tot = loss_coord + loss_conf + loss_cls + loss_theta
    return loss_tot, loss_coord, loss_conf, loss_cls, loss_theta


_a1 = _make_a1()
_stage_c = _make_stage_c()


def kernel(output, target):
    table = jnp.transpose(output, (0, 2, 3, 1)).reshape(_B * _HW, _C)
    t6 = jnp.pad(jnp.transpose(target, (0, 2, 1)), ((0, 0), (0, 0), (0, 12)))
    g = _make_sc_gather()(table, t6)
    confsum = _a1(table)
    lt, lco, lcf, lcl, lth = _stage_c(g, target, confsum)
    return (lt.reshape(()), lco.reshape(()), lcf.reshape(()),
            lcl.reshape(()), lth.reshape(()))
